# baseline (speedup 1.0000x reference)
import functools

import jax
import jax.numpy as jnp
from jax import lax
from jax.experimental import pallas as pl
from jax.experimental.pallas import tpu as pltpu
from jax.experimental.pallas import tpu_sc as plsc

_HIDDEN = 64
_GAMMA = 12.0
_NC = 2
_NS = 16
_NW = _NC * _NS
_CH = 32
_NSEM = 8


def _tec_body(idx_hbm, ent_hbm, rel_hbm, out_hbm,
              idx_v, sbuf, hbuf, rbuf, tbuf, out_v, sems,
              *, rows_per_w, n_chunks):
    sid = lax.axis_index("s")
    wid = sid * _NC + lax.axis_index("c")
    base = wid * rows_per_w

    pltpu.sync_copy(idx_hbm.at[wid], idx_v)

    lane = lax.iota(jnp.int32, 16)

    def fire(c, slot):
        for g in range(_CH // 16):
            rows = (c * _CH + g * 16) + lane
            for t, (tbl, buf) in enumerate(((ent_hbm, hbuf), (rel_hbm, rbuf),
                                            (ent_hbm, tbuf))):
                vec = plsc.load_gather(idx_v, [rows, jnp.full((16,), t,
                                                             jnp.int32)])
                for j in range(16):
                    pltpu.async_copy(tbl.at[vec[j]],
                                     sbuf.at[sid, slot, t, g * 16 + j],
                                     sems.at[(g * 48 + t * 16 + j) % _NSEM])

    def wait(slot):
        for g in range(_CH // 16):
            for t in range(3):
                for j in range(16):
                    pltpu.make_async_copy(
                        ent_hbm.at[0], sbuf.at[sid, slot, t, g * 16 + j],
                        sems.at[(g * 48 + t * 16 + j) % _NSEM]).wait()
        for t, buf in enumerate((hbuf, rbuf, tbuf)):
            pltpu.sync_copy(sbuf.at[sid, slot, t], buf.at[slot])

    def compute(c, slot):
        wait(slot)
        for g in range(_CH // 16):
            rows = g * 16 + lane
            slot_v = jnp.full((16,), slot, jnp.int32)
            acc = None
            for d in range(_HIDDEN):
                dv = jnp.full((16,), d, jnp.int32)
                h = plsc.load_gather(hbuf, [slot_v, rows, dv])
                r = plsc.load_gather(rbuf, [slot_v, rows, dv])
                t = plsc.load_gather(tbuf, [slot_v, rows, dv])
                v = jnp.abs(h + r - t)
                acc = v if acc is None else acc + v
            out_v[pl.ds(c * _CH + g * 16, 16)] = _GAMMA - acc

    fire(0, 0)

    def pair(p, _):
        fire(2 * p + 1, 1)
        compute(2 * p, 0)

        @pl.when(p < n_chunks // 2 - 1)
        def _():
            fire(2 * p + 2, 0)

        compute(2 * p + 1, 1)
        return ()

    lax.fori_loop(0, n_chunks // 2, pair, ())

    pltpu.sync_copy(out_v, out_hbm.at[pl.ds(base, rows_per_w)])


def kernel(sample, entity_embedding, relation_embedding):
    batch = sample.shape[0]
    rows_per_w = batch // _NW
    n_chunks = rows_per_w // _CH

    idx = sample.astype(jnp.int32).reshape(_NW, rows_per_w, 3)

    mesh = plsc.VectorSubcoreMesh(core_axis_name="c", subcore_axis_name="s")
    scores = pl.kernel(
        functools.partial(_tec_body, rows_per_w=rows_per_w, n_chunks=n_chunks),
        out_type=jax.ShapeDtypeStruct((batch,), jnp.float32),
        mesh=mesh,
        compiler_params=pltpu.CompilerParams(needs_layout_passes=False),
        scratch_types=[
            pltpu.VMEM((rows_per_w, 3), jnp.int32),
            pltpu.VMEM_SHARED((_NS, 2, 3, _CH, _HIDDEN), jnp.float32),
            pltpu.VMEM((2, _CH, _HIDDEN), jnp.float32),
            pltpu.VMEM((2, _CH, _HIDDEN), jnp.float32),
            pltpu.VMEM((2, _CH, _HIDDEN), jnp.float32),
            pltpu.VMEM((rows_per_w,), jnp.float32),
            pltpu.SemaphoreType.DMA((_NSEM,)),
        ],
    )(idx, entity_embedding, relation_embedding)
    return scores[:, None]


# speedup vs baseline: 1.0362x; 1.0362x over previous
import functools

import jax
import jax.numpy as jnp
from jax import lax
from jax.experimental import pallas as pl
from jax.experimental.pallas import tpu as pltpu
from jax.experimental.pallas import tpu_sc as plsc

_HIDDEN = 64
_GAMMA = 12.0
_NC = 2
_NS = 16
_NW = _NC * _NS
_CH = 32
_NSEM = 8


def _tec_body(idx_hbm, ent_hbm, rel_hbm, out_hbm,
              idx_v, hbuf, rbuf, tbuf, out_v, sems,
              *, rows_per_w, n_chunks):
    sid = lax.axis_index("s")
    wid = sid * _NC + lax.axis_index("c")
    base = wid * rows_per_w

    pltpu.sync_copy(idx_hbm.at[wid], idx_v)

    lane = lax.iota(jnp.int32, 16)

    def fire(c, slot):
        for g in range(_CH // 16):
            rows = (c * _CH + g * 16) + lane
            for t, (tbl, buf) in enumerate(((ent_hbm, hbuf), (rel_hbm, rbuf),
                                            (ent_hbm, tbuf))):
                vec = plsc.load_gather(idx_v, [rows, jnp.full((16,), t,
                                                             jnp.int32)])
                for j in range(16):
                    i = g * 16 + j
                    pltpu.async_copy(tbl.at[vec[j]],
                                     buf.at[slot, pl.ds(i * _HIDDEN, _HIDDEN)],
                                     sems.at[(g * 48 + t * 16 + j) % _NSEM])

    def wait(slot):
        for g in range(_CH // 16):
            for t, buf in enumerate((hbuf, rbuf, tbuf)):
                for j in range(16):
                    i = g * 16 + j
                    pltpu.make_async_copy(
                        ent_hbm.at[0],
                        buf.at[slot, pl.ds(i * _HIDDEN, _HIDDEN)],
                        sems.at[(g * 48 + t * 16 + j) % _NSEM]).wait()

    def compute(c, slot):
        wait(slot)
        for g in range(_CH // 16):
            wbase = (g * 16 + lane) * _HIDDEN
            slot_v = jnp.full((16,), slot, jnp.int32)
            acc = None
            for d in range(_HIDDEN):
                w = wbase + d
                h = plsc.load_gather(hbuf, [slot_v, w])
                r = plsc.load_gather(rbuf, [slot_v, w])
                t = plsc.load_gather(tbuf, [slot_v, w])
                v = jnp.abs(h + r - t)
                acc = v if acc is None else acc + v
            out_v[pl.ds(c * _CH + g * 16, 16)] = _GAMMA - acc

    fire(0, 0)

    def pair(p, _):
        fire(2 * p + 1, 1)
        compute(2 * p, 0)

        @pl.when(p < n_chunks // 2 - 1)
        def _():
            fire(2 * p + 2, 0)

        compute(2 * p + 1, 1)
        return ()

    lax.fori_loop(0, n_chunks // 2, pair, ())

    pltpu.sync_copy(out_v, out_hbm.at[pl.ds(base, rows_per_w)])


def kernel(sample, entity_embedding, relation_embedding):
    batch = sample.shape[0]
    rows_per_w = batch // _NW
    n_chunks = rows_per_w // _CH

    idx = sample.astype(jnp.int32).reshape(_NW, rows_per_w, 3)

    mesh = plsc.VectorSubcoreMesh(core_axis_name="c", subcore_axis_name="s")
    scores = pl.kernel(
        functools.partial(_tec_body, rows_per_w=rows_per_w, n_chunks=n_chunks),
        out_type=jax.ShapeDtypeStruct((batch,), jnp.float32),
        mesh=mesh,
        compiler_params=pltpu.CompilerParams(needs_layout_passes=False),
        scratch_types=[
            pltpu.VMEM((rows_per_w, 3), jnp.int32),
            pltpu.VMEM((2, _CH * _HIDDEN), jnp.float32),
            pltpu.VMEM((2, _CH * _HIDDEN), jnp.float32),
            pltpu.VMEM((2, _CH * _HIDDEN), jnp.float32),
            pltpu.VMEM((rows_per_w,), jnp.float32),
            pltpu.SemaphoreType.DMA((_NSEM,)),
        ],
    )(idx, entity_embedding, relation_embedding)
    return scores[:, None]


# speedup vs baseline: 1.5766x; 1.5215x over previous
import functools

import jax
import jax.numpy as jnp
from jax import lax
from jax.experimental import pallas as pl
from jax.experimental.pallas import tpu as pltpu
from jax.experimental.pallas import tpu_sc as plsc

_HIDDEN = 64
_GAMMA = 12.0
_NC = 2
_NS = 16
_NW = _NC * _NS
_CH = 32


def _tec_body(idx_hbm, ent3_hbm, rel3_hbm, out_hbm,
              idx_v, hbuf, rbuf, tbuf, out_v, sem,
              *, rows_per_w, n_chunks):
    wid = lax.axis_index("s") * _NC + lax.axis_index("c")
    base = wid * rows_per_w

    pltpu.sync_copy(idx_hbm.at[wid], idx_v)

    lane = lax.iota(jnp.int32, 16)

    def fire(c, slot):
        for g in range(_CH // 16):
            for t, (tbl, buf) in enumerate(((ent3_hbm, hbuf), (rel3_hbm, rbuf),
                                            (ent3_hbm, tbuf))):
                vec = idx_v[t, pl.ds(c * _CH + g * 16, 16)]
                for j in range(16):
                    jj = vec[j]
                    pltpu.async_copy(
                        tbl.at[lax.shift_right_logical(jj, 3), jj & 7],
                        buf.at[slot, g * 16 + j], sem)

    def wait(slot):
        for tbl, buf in ((ent3_hbm, hbuf), (rel3_hbm, rbuf), (ent3_hbm, tbuf)):
            for i in range(_CH):
                pltpu.make_async_copy(tbl.at[0, 0], buf.at[slot, i],
                                      sem).wait()

    def compute(c, slot):
        wait(slot)
        for g in range(_CH // 16):
            rows = g * 16 + lane
            slot_v = jnp.full((16,), slot, jnp.int32)
            acc = None
            for d in range(_HIDDEN):
                dv = jnp.full((16,), d, jnp.int32)
                h = plsc.load_gather(hbuf, [slot_v, rows, dv])
                r = plsc.load_gather(rbuf, [slot_v, rows, dv])
                t = plsc.load_gather(tbuf, [slot_v, rows, dv])
                v = jnp.abs(h + r - t)
                acc = v if acc is None else acc + v
            out_v[pl.ds(c * _CH + g * 16, 16)] = _GAMMA - acc

    fire(0, 0)

    def pair(p, _):
        fire(2 * p + 1, 1)
        compute(2 * p, 0)

        @pl.when(p < n_chunks // 2 - 1)
        def _():
            fire(2 * p + 2, 0)

        compute(2 * p + 1, 1)
        return ()

    lax.fori_loop(0, n_chunks // 2, pair, ())

    pltpu.sync_copy(out_v, out_hbm.at[pl.ds(base, rows_per_w)])


def kernel(sample, entity_embedding, relation_embedding):
    batch = sample.shape[0]
    rows_per_w = batch // _NW
    n_chunks = rows_per_w // _CH

    idx = sample.astype(jnp.int32).T.reshape(3, _NW, rows_per_w)
    idx = idx.transpose(1, 0, 2)
    ent3 = entity_embedding.reshape(-1, 8, _HIDDEN)
    rel3 = relation_embedding.reshape(-1, 8, _HIDDEN)

    mesh = plsc.VectorSubcoreMesh(core_axis_name="c", subcore_axis_name="s")
    scores = pl.kernel(
        functools.partial(_tec_body, rows_per_w=rows_per_w, n_chunks=n_chunks),
        out_type=jax.ShapeDtypeStruct((batch,), jnp.float32),
        mesh=mesh,
        compiler_params=pltpu.CompilerParams(needs_layout_passes=False),
        scratch_types=[
            pltpu.VMEM((3, rows_per_w), jnp.int32),
            pltpu.VMEM((2, _CH, _HIDDEN), jnp.float32),
            pltpu.VMEM((2, _CH, _HIDDEN), jnp.float32),
            pltpu.VMEM((2, _CH, _HIDDEN), jnp.float32),
            pltpu.VMEM((rows_per_w,), jnp.float32),
            pltpu.SemaphoreType.DMA,
        ],
    )(idx, ent3, rel3)
    return scores[:, None]
